# Initial kernel scaffold; baseline (speedup 1.0000x reference)
import jax
import jax.numpy as jnp
from jax.experimental import pallas as pl


def kernel(h, edge_index, edge_attr, fc_w, attn_w):
    raise NotImplementedError("write your pallas kernel here")


# speedup vs baseline: 7.1472x; 7.1472x over previous
import functools

import jax
import jax.numpy as jnp
from jax import lax
from jax.experimental import pallas as pl
from jax.experimental.pallas import tpu as pltpu
from jax.experimental.pallas import tpu_sc as plsc

N = 10000
NP = 10240
E = 160000
D_IN = 256
D_OUT = 256
D_EDGE = 16
HALF = 128
NEG_SLOPE = 0.2

NSUB = 16
EPT = E // NSUB
CHUNK = 80
SUPER = 5
SCC = 25
ROWS_PT = NP // NSUB

_f32 = jnp.float32


_BN = 400


def _proj_body(h_ref, fcw_ref, w12_ref, z2_ref, st_ref):
    z = lax.dot_general(h_ref[...], fcw_ref[...], (((1,), (1,)), ((), ())),
                        preferred_element_type=_f32,
                        precision=lax.Precision.HIGHEST)
    z2_ref[0] = z[:, :HALF]
    z2_ref[1] = z[:, HALF:]
    st_ref[...] = lax.dot_general(z, w12_ref[...], (((1,), (0,)), ((), ())),
                                  preferred_element_type=_f32,
                                  precision=lax.Precision.HIGHEST)


_proj = pl.pallas_call(
    _proj_body,
    grid=(N // _BN,),
    in_specs=[pl.BlockSpec((_BN, D_IN), lambda i: (i, 0)),
              pl.BlockSpec((D_OUT, D_IN), lambda i: (0, 0)),
              pl.BlockSpec((D_OUT, 8), lambda i: (0, 0))],
    out_specs=[pl.BlockSpec((2, _BN, HALF), lambda i: (0, i, 0)),
               pl.BlockSpec((_BN, 8), lambda i: (i, 0))],
    out_shape=[jax.ShapeDtypeStruct((2, N, HALF), _f32),
               jax.ShapeDtypeStruct((N, 8), _f32)],
)


_BE = 2000


def _edge_body(ea_ref, w3_ref, g_ref):
    g_ref[...] = lax.dot_general(ea_ref[...], w3_ref[...],
                                 (((1,), (0,)), ((), ())),
                                 preferred_element_type=_f32,
                                 precision=lax.Precision.HIGHEST)


_edge = pl.pallas_call(
    _edge_body,
    grid=(E // _BE,),
    in_specs=[pl.BlockSpec((_BE, D_EDGE), lambda i: (i, 0)),
              pl.BlockSpec((D_EDGE, 8), lambda i: (0, 0))],
    out_specs=pl.BlockSpec((_BE, 8), lambda i: (i, 0)),
    out_shape=jax.ShapeDtypeStruct((E, 8), _f32),
)


_BC = 80


def _norm_body(ul_ref, ur_ref, den_ref, out_ref):
    den = den_ref[...]
    den = jnp.where(den == 0.0, 1.0, den)
    out_ref[:, :HALF] = ul_ref[...] / den
    out_ref[:, HALF:] = ur_ref[...] / den


_norm = pl.pallas_call(
    _norm_body,
    grid=(N // _BC,),
    in_specs=[pl.BlockSpec((_BC, HALF), lambda i: (i, 0)),
              pl.BlockSpec((_BC, HALF), lambda i: (i + NP // _BC, 0)),
              pl.BlockSpec((_BC, 1), lambda i: (i, 0))],
    out_specs=pl.BlockSpec((_BC, D_OUT), lambda i: (i, 0)),
    out_shape=jax.ShapeDtypeStruct((N, D_OUT), _f32),
)


_sc_mesh = plsc.VectorSubcoreMesh(core_axis_name="c", subcore_axis_name="s")


@functools.partial(
    pl.kernel,
    out_type=(jax.ShapeDtypeStruct((2 * NP, HALF), _f32),
              jax.ShapeDtypeStruct((NSUB, ROWS_PT), _f32)),
    mesh=_sc_mesh,
    compiler_params=pltpu.CompilerParams(needs_layout_passes=False,
                                         use_tc_tiling_on_sc=False),
    scratch_types=[
        pltpu.VMEM((N,), _f32),
        pltpu.VMEM((N,), _f32),
        pltpu.VMEM((SCC, CHUNK), jnp.int32),
        pltpu.VMEM((SCC, CHUNK), jnp.int32),
        pltpu.VMEM((SCC, CHUNK), _f32),
        pltpu.VMEM((CHUNK, HALF), _f32),
        pltpu.VMEM((CHUNK, HALF), _f32),
        pltpu.VMEM((CHUNK,), _f32),
        pltpu.VMEM((16,), _f32),
        pltpu.VMEM((NSUB, 16), _f32),
        pltpu.VMEM((ROWS_PT,), _f32),
        pltpu.VMEM_SHARED((NP, HALF), _f32),
        pltpu.VMEM_SHARED((NP,), _f32),
        pltpu.VMEM_SHARED((NSUB, 16), _f32),
        pltpu.SemaphoreType.DMA,
    ],
)
def _sc_gat(z2f, s1, t1, g4, src4, dst4, u2, den_out,
            s_v, t_v, src_sb, dst_sb, g_sb, gbuf0, gbuf1, exb,
            mxv, mx16, denb, u_sh, den_f, max_sh, gsem):
    cid = lax.axis_index("c")
    sid = lax.axis_index("s")
    rbase = pl.multiple_of(sid * ROWS_PT, ROWS_PT)
    zeros16i = jnp.zeros((16,), jnp.int32)
    ones16i = jnp.ones((16,), jnp.int32)
    zeros16f = jnp.zeros((16,), _f32)
    zoffv = jnp.full((16,), cid * N, jnp.int32)

    pltpu.sync_copy(s1, s_v)
    pltpu.sync_copy(t1, t_v)

    def _zg(r, carry):
        for k in range(HALF // 16):
            gbuf0[r, pl.ds(k * 16, 16)] = zeros16f
        return carry
    lax.fori_loop(0, CHUNK, _zg, None)
    for j in range(CHUNK // 16):
        exb[pl.ds(j * 16, 16)] = zeros16f

    def _zu(i, carry):
        pltpu.sync_copy(gbuf0, u_sh.at[pl.ds(rbase + i * CHUNK, CHUNK)])
        pltpu.sync_copy(exb, den_f.at[pl.ds(rbase + i * CHUNK, CHUNK)])
        return carry
    lax.fori_loop(0, ROWS_PT // CHUNK, _zu, None)

    def _p1(s, mx):
        pltpu.sync_copy(src4.at[sid, s], src_sb)
        pltpu.sync_copy(dst4.at[sid, s], dst_sb)
        pltpu.sync_copy(g4.at[sid, s], g_sb)

        def _c(c, mxc):
            for j in range(CHUNK // 16):
                sl = pl.ds(j * 16, 16)
                s_ = plsc.load_gather(s_v, [src_sb[c, sl]])
                t_ = plsc.load_gather(t_v, [dst_sb[c, sl]])
                a = s_ + t_ + g_sb[c, sl]
                e = jnp.where(a >= 0.0, a, a * NEG_SLOPE)
                mxc = jnp.maximum(mxc, e)
            return mxc
        return lax.fori_loop(0, SCC, _c, mx)
    mx = lax.fori_loop(0, SUPER, _p1, jnp.full((16,), -jnp.inf, _f32))

    mxv[...] = mx
    pltpu.sync_copy(mxv, max_sh.at[sid])
    plsc.subcore_barrier()
    pltpu.sync_copy(max_sh, mx16)

    def _mr(i, m):
        return jnp.maximum(m, mx16[i])
    mx = lax.fori_loop(0, NSUB, _mr, jnp.full((16,), -jnp.inf, _f32))
    gmaxv = jnp.full((16,), jnp.max(mx), _f32)

    def _chunk(gcur, gnxt, c, prefetch):
        pltpu.make_async_copy(z2f.at[src_sb.at[c]], gcur, gsem).wait()
        if prefetch:
            pltpu.async_copy(z2f.at[src_sb.at[c + 1]], gnxt, gsem)
        for j in range(CHUNK // 16):
            sl = pl.ds(j * 16, 16)
            sv = src_sb[c, sl] - zoffv
            s_ = plsc.load_gather(s_v, [sv])
            t_ = plsc.load_gather(t_v, [dst_sb[c, sl]])
            a = s_ + t_ + g_sb[c, sl]
            e = jnp.where(a >= 0.0, a, a * NEG_SLOPE)
            exb[sl] = jnp.exp(e - gmaxv)
        pltpu.sync_copy(exb, den_f.at[dst_sb.at[c]], add=True)

        def _srow(r, carry):
            exs = plsc.load_gather(exb, [jnp.full((16,), r, jnp.int32)])
            for k in range(HALF // 16):
                slk = pl.ds(k * 16, 16)
                gcur[r, slk] = gcur[r, slk] * exs
            return carry
        lax.fori_loop(0, CHUNK, _srow, None)
        pltpu.sync_copy(gcur, u_sh.at[dst_sb.at[c]], add=True)

    def _p2(s, carry):
        pltpu.sync_copy(src4.at[sid, s], src_sb)
        pltpu.sync_copy(dst4.at[sid, s], dst_sb)
        pltpu.sync_copy(g4.at[sid, s], g_sb)

        def _tr(c, carry2):
            for j in range(CHUNK // 16):
                sl = pl.ds(j * 16, 16)
                src_sb[c, sl] = src_sb[c, sl] + zoffv
            return carry2
        lax.fori_loop(0, SCC, _tr, None)

        pltpu.async_copy(z2f.at[src_sb.at[0]], gbuf0, gsem)

        def _pair(p, carry2):
            _chunk(gbuf0, gbuf1, 2 * p, True)
            _chunk(gbuf1, gbuf0, 2 * p + 1, True)
            return carry2
        lax.fori_loop(0, SCC // 2, _pair, None)
        _chunk(gbuf0, gbuf1, SCC - 1, False)
        return carry
    lax.fori_loop(0, SUPER, _p2, None)

    plsc.subcore_barrier()

    pltpu.sync_copy(den_f.at[pl.ds(rbase, ROWS_PT)], denb)

    @pl.when(cid == 0)
    def _():
        pltpu.sync_copy(denb, den_out.at[sid])

    ubase = cid * NP + rbase
    def _wo(i, carry):
        pltpu.sync_copy(u_sh.at[pl.ds(rbase + i * CHUNK, CHUNK)], gbuf0)
        pltpu.sync_copy(gbuf0, u2.at[pl.ds(ubase + i * CHUNK, CHUNK)])
        return carry
    lax.fori_loop(0, ROWS_PT // CHUNK, _wo, None)


def kernel(h, edge_index, edge_attr, fc_w, attn_w):
    w = attn_w[0]
    w12 = jnp.zeros((D_OUT, 8), _f32)
    w12 = w12.at[:, 0].set(w[:D_OUT]).at[:, 1].set(w[D_OUT:2 * D_OUT])
    w3 = jnp.zeros((D_EDGE, 8), _f32).at[:, 0].set(w[2 * D_OUT:])

    z2, st8 = _proj(h, fc_w, w12)
    g8 = _edge(edge_attr, w3)

    z2f = z2.reshape(2 * N, HALF)
    s1 = st8[:, 0]
    t1 = st8[:, 1]
    g4 = g8[:, 0].reshape(NSUB, SUPER, SCC, CHUNK)
    src4 = edge_index[0].reshape(NSUB, SUPER, SCC, CHUNK)
    dst4 = edge_index[1].reshape(NSUB, SUPER, SCC, CHUNK)

    u2, den = _sc_gat(z2f, s1, t1, g4, src4, dst4)
    return _norm(u2, u2, den.reshape(NP, 1))
